# Initial kernel scaffold; baseline (speedup 1.0000x reference)
import jax
import jax.numpy as jnp
from jax.experimental import pallas as pl


def kernel(h_node, pos_node, batch_node, h_edge, edge_index, batch_edge, params):
    raise NotImplementedError("write your pallas kernel here")


# speedup vs baseline: 1.0231x; 1.0231x over previous
import functools

import jax
import jax.numpy as jnp
from jax import lax
from jax.experimental import pallas as pl
from jax.experimental.pallas import tpu as pltpu
from jax.experimental.pallas import tpu_sc as plsc

N = 10000
E = 320000
G = 64

TN = 400
TEE = 1280
TEP = 2000
ECH = 8000

_f32 = jnp.float32


def _dot(a, b):
    return jnp.dot(a, b, preferred_element_type=_f32)


def _node0_body(hnode_ref, wemb_ref, pac_ref, pb_ref, hn_ref, ac_ref, b_ref):
    hn = _dot(hnode_ref[...], wemb_ref[...])
    hn_ref[...] = hn
    ac_ref[...] = _dot(hn, pac_ref[...])
    b_ref[...] = _dot(hn, pb_ref[...])


def _node0(h_node, wemb, pac, pb):
    nt = N // TN
    full = lambda i: (0, 0)
    return pl.pallas_call(
        _node0_body,
        grid=(nt,),
        in_specs=[
            pl.BlockSpec((TN, 16), lambda i: (i, 0)),
            pl.BlockSpec((16, 128), full),
            pl.BlockSpec((128, 256), full),
            pl.BlockSpec((128, 128), full),
        ],
        out_specs=[
            pl.BlockSpec((TN, 128), lambda i: (i, 0)),
            pl.BlockSpec((TN, 256), lambda i: (i, 0)),
            pl.BlockSpec((TN, 128), lambda i: (i, 0)),
        ],
        out_shape=[
            jax.ShapeDtypeStruct((N, 128), _f32),
            jax.ShapeDtypeStruct((N, 256), _f32),
            jax.ShapeDtypeStruct((N, 128), _f32),
        ],
    )(h_node, wemb, pac, pb)


def _prep_body(ps_ref, pd_ref, hep_ref, wee_ref, dist_ref, he0_ref):
    d = ps_ref[...] - pd_ref[...]
    s = jnp.sum(d * d, axis=1, keepdims=True)
    dist_ref[...] = jnp.broadcast_to(jnp.sqrt(s), (TEP, 8))
    he0_ref[...] = _dot(hep_ref[...], wee_ref[...])


def _prep(ps, pd, hep, wee):
    nt = E // TEP
    return pl.pallas_call(
        _prep_body,
        grid=(nt,),
        in_specs=[
            pl.BlockSpec((TEP, 16), lambda i: (i, 0)),
            pl.BlockSpec((TEP, 16), lambda i: (i, 0)),
            pl.BlockSpec((TEP, 8), lambda i: (i, 0)),
            pl.BlockSpec((8, 64), lambda i: (0, 0)),
        ],
        out_specs=[
            pl.BlockSpec((TEP, 8), lambda i: (i, 0)),
            pl.BlockSpec((TEP, 64), lambda i: (i, 0)),
        ],
        out_shape=[
            jax.ShapeDtypeStruct((E, 8), _f32),
            jax.ShapeDtypeStruct((E, 64), _f32),
        ],
    )(ps, pd, hep, wee)


def _edge_body(ac_ref, b_ref, he_ref, dist_ref, w1e_ref, w1v_ref, be1_ref,
               we2_ref, be2_ref, wme_ref, bm_ref, heo_ref, msg_ref):
    ac = ac_ref[...]
    dist = dist_ref[...][:, :1]
    t = ac[:, :128] + b_ref[...] + _dot(he_ref[...], w1e_ref[...])
    t = t + dist * w1v_ref[...] + be1_ref[...]
    t = jnp.maximum(t, 0.0)
    he2 = _dot(t, we2_ref[...]) + be2_ref[...]
    heo_ref[...] = he2
    msg = ac[:, 128:] + _dot(he2, wme_ref[...]) + bm_ref[...]
    msg_ref[...] = jnp.maximum(msg, 0.0)


def _edge(ac_src, b_dst, he, dist, w1e, w1v, be1, we2, be2, wme, bm):
    nt = E // TEE
    full = lambda i: (0, 0)
    return pl.pallas_call(
        _edge_body,
        grid=(nt,),
        in_specs=[
            pl.BlockSpec((TEE, 256), lambda i: (i, 0)),
            pl.BlockSpec((TEE, 128), lambda i: (i, 0)),
            pl.BlockSpec((TEE, 64), lambda i: (i, 0)),
            pl.BlockSpec((TEE, 8), lambda i: (i, 0)),
            pl.BlockSpec((64, 128), full),
            pl.BlockSpec((1, 128), full),
            pl.BlockSpec((1, 128), full),
            pl.BlockSpec((128, 64), full),
            pl.BlockSpec((1, 64), full),
            pl.BlockSpec((64, 128), full),
            pl.BlockSpec((1, 128), full),
        ],
        out_specs=[
            pl.BlockSpec((TEE, 64), lambda i: (i, 0)),
            pl.BlockSpec((TEE, 128), lambda i: (i, 0)),
        ],
        out_shape=[
            jax.ShapeDtypeStruct((E, 64), _f32),
            jax.ShapeDtypeStruct((E, 128), _f32),
        ],
    )(ac_src, b_dst, he, dist, w1e, w1v, be1, we2, be2, wme, bm)


def _upd_body(hn_ref, a0_ref, a1_ref, wnh_ref, wna_ref, bn_ref, pac_ref,
              pb_ref, hn1_ref, ac_ref, b_ref):
    hn = hn_ref[...]
    agg = a0_ref[...] + a1_ref[...]
    hn1 = hn + _dot(hn, wnh_ref[...]) + _dot(agg, wna_ref[...]) + bn_ref[...]
    hn1_ref[...] = hn1
    ac_ref[...] = _dot(hn1, pac_ref[...])
    b_ref[...] = _dot(hn1, pb_ref[...])


def _update(hn, agg0, agg1, wnh, wna, bn, pac, pb):
    nt = N // TN
    full = lambda i: (0, 0)
    return pl.pallas_call(
        _upd_body,
        grid=(nt,),
        in_specs=[
            pl.BlockSpec((TN, 128), lambda i: (i, 0)),
            pl.BlockSpec((TN, 128), lambda i: (i, 0)),
            pl.BlockSpec((TN, 128), lambda i: (i, 0)),
            pl.BlockSpec((128, 128), full),
            pl.BlockSpec((128, 128), full),
            pl.BlockSpec((1, 128), full),
            pl.BlockSpec((128, 256), full),
            pl.BlockSpec((128, 128), full),
        ],
        out_specs=[
            pl.BlockSpec((TN, 128), lambda i: (i, 0)),
            pl.BlockSpec((TN, 256), lambda i: (i, 0)),
            pl.BlockSpec((TN, 128), lambda i: (i, 0)),
        ],
        out_shape=[
            jax.ShapeDtypeStruct((N, 128), _f32),
            jax.ShapeDtypeStruct((N, 256), _f32),
            jax.ShapeDtypeStruct((N, 128), _f32),
        ],
    )(hn, agg0, agg1, wnh, wna, bn, pac, pb)


def _final_body(hn_ref, bn3_ref, he_ref, be3_ref, w1_ref, b1_ref, w2_ref,
                b2_ref, auxw_ref, emb_ref, predp_ref, ns_s, nc_s, es_s, ec_s):
    i = pl.program_id(0)
    nsteps = pl.num_programs(0)

    @pl.when(i == 0)
    def _():
        bid = bn3_ref[...].reshape(1, N)
        gid = lax.broadcasted_iota(jnp.int32, (G, N), 0)
        oh = (gid == bid).astype(_f32)
        ns_s[...] = _dot(oh, hn_ref[...])
        nc_s[...] = jnp.broadcast_to(jnp.sum(oh, axis=1, keepdims=True), (G, 128))
        es_s[...] = jnp.zeros((G, 64), _f32)
        ec_s[...] = jnp.zeros((G, 128), _f32)

    bide = be3_ref[...].reshape(1, ECH)
    gide = lax.broadcasted_iota(jnp.int32, (G, ECH), 0)
    ohe = (gide == bide).astype(_f32)
    es_s[...] += _dot(ohe, he_ref[...])
    ec_s[...] += jnp.broadcast_to(jnp.sum(ohe, axis=1, keepdims=True), (G, 128))

    @pl.when(i == nsteps - 1)
    def _():
        nmean = ns_s[...] / jnp.maximum(nc_s[...], 1.0)
        emean = es_s[...] / jnp.maximum(ec_s[...][:, :64], 1.0)
        hsub = jnp.concatenate([nmean, emean], axis=1)
        z = jnp.maximum(_dot(hsub, w1_ref[...]) + b1_ref[...], 0.0)
        emb = _dot(z, w2_ref[...]) + b2_ref[...]
        emb_ref[...] = emb
        predp_ref[...] = _dot(emb[:, :64], auxw_ref[...])


def _final(hn, bn3, he, be3, w1, b1, w2, b2, auxw):
    nt = E // ECH
    full2 = lambda i: (0, 0)
    return pl.pallas_call(
        _final_body,
        grid=(nt,),
        in_specs=[
            pl.BlockSpec((N, 128), full2),
            pl.BlockSpec((1, 1, N), lambda i: (0, 0, 0)),
            pl.BlockSpec((ECH, 64), lambda i: (i, 0)),
            pl.BlockSpec((1, 1, ECH), lambda i: (i, 0, 0)),
            pl.BlockSpec((192, 256), full2),
            pl.BlockSpec((1, 256), full2),
            pl.BlockSpec((256, 128), full2),
            pl.BlockSpec((1, 128), full2),
            pl.BlockSpec((64, 128), full2),
        ],
        out_specs=[
            pl.BlockSpec((G, 128), full2),
            pl.BlockSpec((G, 128), full2),
        ],
        out_shape=[
            jax.ShapeDtypeStruct((G, 128), _f32),
            jax.ShapeDtypeStruct((G, 128), _f32),
        ],
        scratch_shapes=[
            pltpu.VMEM((G, 128), _f32),
            pltpu.VMEM((G, 128), _f32),
            pltpu.VMEM((G, 64), _f32),
            pltpu.VMEM((G, 128), _f32),
        ],
    )(hn, bn3, he, be3, w1, b1, w2, b2, auxw)


def kernel(h_node, pos_node, batch_node, h_edge, edge_index, batch_edge, params):
    src = edge_index[0]
    dst = edge_index[1]

    pos_pad = jnp.pad(pos_node.astype(_f32), ((0, 0), (0, 13)))
    hep = jnp.pad(h_edge.astype(_f32), ((0, 0), (0, 3)))
    weep = jnp.pad(params['edge_emb'].astype(_f32), ((0, 3), (0, 0)))

    blocks = params['blocks']
    pacs, pbs = [], []
    for blk in blocks:
        w1 = blk['We1']
        pacs.append(jnp.concatenate([w1[:128], blk['Wm'][:128]], axis=1))
        pbs.append(w1[128:256])

    hn, ac, b = _node0(h_node.astype(_f32), params['node_emb'], pacs[0], pbs[0])

    ps = pos_pad[src]
    pd = pos_pad[dst]
    dist, he = _prep(ps, pd, hep, weep)

    for bi, blk in enumerate(blocks):
        w1 = blk['We1']
        ac_src = ac[src]
        b_dst = b[dst]
        he, msg = _edge(
            ac_src, b_dst, he, dist,
            w1[256:320], w1[320:321], blk['be1'].reshape(1, 128),
            blk['We2'], blk['be2'].reshape(1, 64),
            blk['Wm'][128:], blk['bm'].reshape(1, 128),
        )
        agg = jax.ops.segment_sum(msg, dst, num_segments=N)
        zero = jnp.zeros((N, 128), _f32)
        nb = min(bi + 1, 2)
        hn, ac, b = _update(hn, agg, zero, blk['Wn'][:128], blk['Wn'][128:],
                            blk['bn'].reshape(1, 128), pacs[nb], pbs[nb])

    bn3 = batch_node.astype(jnp.int32).reshape(1, 1, N)
    be3 = batch_edge.astype(jnp.int32).reshape(E // ECH, 1, ECH)
    auxwp = jnp.pad(params['aux_W'].astype(_f32), ((0, 0), (0, 127)))
    emb, predp = _final(hn, bn3, he, be3,
                        params['final_W1'], params['final_b1'].reshape(1, 256),
                        params['final_W2'], params['final_b2'].reshape(1, 128),
                        auxwp)
    pred = predp[:, :1] + params['aux_b'].reshape(1, 1)
    return (emb, batch_node, pred)


# speedup vs baseline: 2.8507x; 2.7863x over previous
import functools

import jax
import jax.numpy as jnp
from jax import lax
from jax.experimental import pallas as pl
from jax.experimental.pallas import tpu as pltpu
from jax.experimental.pallas import tpu_sc as plsc

N = 10000
E = 320000
G = 64

NW = 32
EPW = E // NW
GCH = 40
NCH = EPW // GCH
RING = 5

TN = 400
TEE = 1280
TEP = 2000
ECH = 8000

_f32 = jnp.float32


def _dot(a, b):
    return jnp.dot(a, b, preferred_element_type=_f32)


def _make_gather2(d1, d2):
    mesh = plsc.VectorSubcoreMesh(core_axis_name="c", subcore_axis_name="s")

    @functools.partial(
        pl.kernel,
        out_type=[jax.ShapeDtypeStruct((E, d1), _f32),
                  jax.ShapeDtypeStruct((E, d2), _f32)],
        mesh=mesh,
        scratch_types=[
            pltpu.VMEM((EPW,), jnp.int32),
            pltpu.VMEM((EPW,), jnp.int32),
            pltpu.VMEM((RING, GCH, d1), _f32),
            pltpu.VMEM((RING, GCH, d2), _f32),
            pltpu.SemaphoreType.DMA((RING,)),
            pltpu.SemaphoreType.DMA((RING,)),
            pltpu.SemaphoreType.DMA((RING,)),
            pltpu.SemaphoreType.DMA((RING,)),
        ])
    def kern(i1r, i2r, t1r, t2r, o1r, o2r, i1v, i2v, b1, b2, g1s, g2s,
             w1s, w2s):
        cid = lax.axis_index("c")
        sid = lax.axis_index("s")
        wid = sid * 2 + cid
        base = wid * EPW
        pltpu.sync_copy(i1r.at[pl.ds(base, EPW)], i1v)
        pltpu.sync_copy(i2r.at[pl.ds(base, EPW)], i2v)

        def fire(i, b):
            pltpu.async_copy(t1r.at[i1v.at[pl.ds(i * GCH, GCH)]], b1.at[b],
                             g1s.at[b])
            pltpu.async_copy(t2r.at[i2v.at[pl.ds(i * GCH, GCH)]], b2.at[b],
                             g2s.at[b])

        for b in range(RING):
            fire(b, b)

        def step(s, carry):
            for b in range(RING):
                i = s * RING + b
                off = base + i * GCH
                pltpu.make_async_copy(t1r.at[pl.ds(0, GCH)], b1.at[b],
                                      g1s.at[b]).wait()
                c1 = pltpu.async_copy(b1.at[b], o1r.at[pl.ds(off, GCH)],
                                      w1s.at[b])
                pltpu.make_async_copy(t2r.at[pl.ds(0, GCH)], b2.at[b],
                                      g2s.at[b]).wait()
                c2 = pltpu.async_copy(b2.at[b], o2r.at[pl.ds(off, GCH)],
                                      w2s.at[b])
                c1.wait()
                c2.wait()

                @pl.when(i + RING < NCH)
                def _():
                    fire(i + RING, b)
            return carry

        lax.fori_loop(0, NCH // RING, step, 0)

    return kern


_G_ACB = _make_gather2(256, 128)


_G_POS = _make_gather2(128, 128)


_MCH = 40
_MRING = 2
_NPT = N // 16

_sc_mesh = plsc.VectorSubcoreMesh(core_axis_name="c", subcore_axis_name="s")


@functools.partial(
    pl.kernel,
    out_type=jax.ShapeDtypeStruct((2, N, 128), _f32),
    mesh=_sc_mesh,
    scratch_types=[
        pltpu.VMEM((EPW // GCH, 1, GCH), jnp.int32),
        pltpu.VMEM((_MRING, _MCH, 128), _f32),
        pltpu.VMEM((16, 128), _f32),
        pltpu.VMEM_SHARED((N, 128), _f32),
        pltpu.SemaphoreType.DMA((_MRING,)),
    ])
def _sc_scatter(dst4r, msgr, outr, idxv, mb, zb, acc, gs):
    cid = lax.axis_index("c")
    sid = lax.axis_index("s")
    wid = sid * 2 + cid

    for r in range(16):
        for c in range(8):
            zb[r, pl.ds(c * 16, 16)] = jnp.zeros((16,), _f32)
    for k in range(40):
        pltpu.sync_copy(zb, acc.at[pl.ds(sid * 624 + k * 16, 16)])
    plsc.subcore_barrier()

    pltpu.sync_copy(dst4r.at[wid], idxv)

    def fire(i, b):
        pltpu.async_copy(msgr.at[pl.ds(wid * EPW + i * _MCH, _MCH)], mb.at[b],
                         gs.at[b])

    for b in range(_MRING):
        fire(b, b)

    nmch = EPW // _MCH

    def step(s, carry):
        for b in range(_MRING):
            i = s * _MRING + b
            pltpu.make_async_copy(msgr.at[pl.ds(0, _MCH)], mb.at[b],
                                  gs.at[b]).wait()
            for j in range(_MCH // GCH):
                pltpu.sync_copy(mb.at[b, pl.ds(j * GCH, GCH)],
                                acc.at[idxv.at[i * (_MCH // GCH) + j, 0]],
                                add=True)

            @pl.when(i + _MRING < nmch)
            def _():
                fire(i + _MRING, b)
        return carry

    lax.fori_loop(0, nmch // _MRING, step, 0)
    plsc.subcore_barrier()

    @pl.when(sid < 15)
    def _():
        pltpu.sync_copy(acc.at[pl.ds(sid * 624, 624)],
                        outr.at[cid, pl.ds(sid * 624, 624)])

    @pl.when(sid == 15)
    def _():
        pltpu.sync_copy(acc.at[pl.ds(15 * 624, 640)],
                        outr.at[cid, pl.ds(15 * 624, 640)])


def _node0_body(hnode_ref, wemb_ref, pac_ref, pb_ref, hn_ref, ac_ref, b_ref):
    hn = _dot(hnode_ref[...], wemb_ref[...])
    hn_ref[...] = hn
    ac_ref[...] = _dot(hn, pac_ref[...])
    b_ref[...] = _dot(hn, pb_ref[...])


def _node0(h_node, wemb, pac, pb):
    nt = N // TN
    full = lambda i: (0, 0)
    return pl.pallas_call(
        _node0_body,
        grid=(nt,),
        in_specs=[
            pl.BlockSpec((TN, 16), lambda i: (i, 0)),
            pl.BlockSpec((16, 128), full),
            pl.BlockSpec((128, 256), full),
            pl.BlockSpec((128, 128), full),
        ],
        out_specs=[
            pl.BlockSpec((TN, 128), lambda i: (i, 0)),
            pl.BlockSpec((TN, 256), lambda i: (i, 0)),
            pl.BlockSpec((TN, 128), lambda i: (i, 0)),
        ],
        out_shape=[
            jax.ShapeDtypeStruct((N, 128), _f32),
            jax.ShapeDtypeStruct((N, 256), _f32),
            jax.ShapeDtypeStruct((N, 128), _f32),
        ],
    )(h_node, wemb, pac, pb)


def _prep_body(ps_ref, pd_ref, hep_ref, wee_ref, dist_ref, he0_ref):
    d = ps_ref[...] - pd_ref[...]
    s = jnp.sum(d * d, axis=1, keepdims=True)
    dist_ref[...] = jnp.broadcast_to(jnp.sqrt(s), (TEP, 8))
    he0_ref[...] = _dot(hep_ref[...], wee_ref[...])


def _prep(ps, pd, hep, wee):
    nt = E // TEP
    return pl.pallas_call(
        _prep_body,
        grid=(nt,),
        in_specs=[
            pl.BlockSpec((TEP, 128), lambda i: (i, 0)),
            pl.BlockSpec((TEP, 128), lambda i: (i, 0)),
            pl.BlockSpec((TEP, 8), lambda i: (i, 0)),
            pl.BlockSpec((8, 64), lambda i: (0, 0)),
        ],
        out_specs=[
            pl.BlockSpec((TEP, 8), lambda i: (i, 0)),
            pl.BlockSpec((TEP, 64), lambda i: (i, 0)),
        ],
        out_shape=[
            jax.ShapeDtypeStruct((E, 8), _f32),
            jax.ShapeDtypeStruct((E, 64), _f32),
        ],
    )(ps, pd, hep, wee)


def _edge_body(ac_ref, b_ref, he_ref, dist_ref, w1e_ref, w1v_ref, be1_ref,
               we2_ref, be2_ref, wme_ref, bm_ref, heo_ref, msg_ref):
    ac = ac_ref[...]
    dist = dist_ref[...][:, :1]
    t = ac[:, :128] + b_ref[...] + _dot(he_ref[...], w1e_ref[...])
    t = t + dist * w1v_ref[...] + be1_ref[...]
    t = jnp.maximum(t, 0.0)
    he2 = _dot(t, we2_ref[...]) + be2_ref[...]
    heo_ref[...] = he2
    msg = ac[:, 128:] + _dot(he2, wme_ref[...]) + bm_ref[...]
    msg_ref[...] = jnp.maximum(msg, 0.0)


def _edge(ac_src, b_dst, he, dist, w1e, w1v, be1, we2, be2, wme, bm):
    nt = E // TEE
    full = lambda i: (0, 0)
    return pl.pallas_call(
        _edge_body,
        grid=(nt,),
        in_specs=[
            pl.BlockSpec((TEE, 256), lambda i: (i, 0)),
            pl.BlockSpec((TEE, 128), lambda i: (i, 0)),
            pl.BlockSpec((TEE, 64), lambda i: (i, 0)),
            pl.BlockSpec((TEE, 8), lambda i: (i, 0)),
            pl.BlockSpec((64, 128), full),
            pl.BlockSpec((1, 128), full),
            pl.BlockSpec((1, 128), full),
            pl.BlockSpec((128, 64), full),
            pl.BlockSpec((1, 64), full),
            pl.BlockSpec((64, 128), full),
            pl.BlockSpec((1, 128), full),
        ],
        out_specs=[
            pl.BlockSpec((TEE, 64), lambda i: (i, 0)),
            pl.BlockSpec((TEE, 128), lambda i: (i, 0)),
        ],
        out_shape=[
            jax.ShapeDtypeStruct((E, 64), _f32),
            jax.ShapeDtypeStruct((E, 128), _f32),
        ],
    )(ac_src, b_dst, he, dist, w1e, w1v, be1, we2, be2, wme, bm)


def _upd_body(hn_ref, a0_ref, a1_ref, wnh_ref, wna_ref, bn_ref, pac_ref,
              pb_ref, hn1_ref, ac_ref, b_ref):
    hn = hn_ref[...]
    agg = a0_ref[...] + a1_ref[...]
    hn1 = hn + _dot(hn, wnh_ref[...]) + _dot(agg, wna_ref[...]) + bn_ref[...]
    hn1_ref[...] = hn1
    ac_ref[...] = _dot(hn1, pac_ref[...])
    b_ref[...] = _dot(hn1, pb_ref[...])


def _update(hn, agg0, agg1, wnh, wna, bn, pac, pb):
    nt = N // TN
    full = lambda i: (0, 0)
    return pl.pallas_call(
        _upd_body,
        grid=(nt,),
        in_specs=[
            pl.BlockSpec((TN, 128), lambda i: (i, 0)),
            pl.BlockSpec((TN, 128), lambda i: (i, 0)),
            pl.BlockSpec((TN, 128), lambda i: (i, 0)),
            pl.BlockSpec((128, 128), full),
            pl.BlockSpec((128, 128), full),
            pl.BlockSpec((1, 128), full),
            pl.BlockSpec((128, 256), full),
            pl.BlockSpec((128, 128), full),
        ],
        out_specs=[
            pl.BlockSpec((TN, 128), lambda i: (i, 0)),
            pl.BlockSpec((TN, 256), lambda i: (i, 0)),
            pl.BlockSpec((TN, 128), lambda i: (i, 0)),
        ],
        out_shape=[
            jax.ShapeDtypeStruct((N, 128), _f32),
            jax.ShapeDtypeStruct((N, 256), _f32),
            jax.ShapeDtypeStruct((N, 128), _f32),
        ],
    )(hn, agg0, agg1, wnh, wna, bn, pac, pb)


def _final_body(hn_ref, bn3_ref, he_ref, be3_ref, w1_ref, b1_ref, w2_ref,
                b2_ref, auxw_ref, emb_ref, predp_ref, ns_s, nc_s, es_s, ec_s):
    i = pl.program_id(0)
    nsteps = pl.num_programs(0)

    @pl.when(i == 0)
    def _():
        bid = bn3_ref[...].reshape(1, N)
        gid = lax.broadcasted_iota(jnp.int32, (G, N), 0)
        oh = (gid == bid).astype(_f32)
        ns_s[...] = _dot(oh, hn_ref[...])
        nc_s[...] = jnp.broadcast_to(jnp.sum(oh, axis=1, keepdims=True), (G, 128))
        es_s[...] = jnp.zeros((G, 64), _f32)
        ec_s[...] = jnp.zeros((G, 128), _f32)

    bide = be3_ref[...].reshape(1, ECH)
    gide = lax.broadcasted_iota(jnp.int32, (G, ECH), 0)
    ohe = (gide == bide).astype(_f32)
    es_s[...] += _dot(ohe, he_ref[...])
    ec_s[...] += jnp.broadcast_to(jnp.sum(ohe, axis=1, keepdims=True), (G, 128))

    @pl.when(i == nsteps - 1)
    def _():
        nmean = ns_s[...] / jnp.maximum(nc_s[...], 1.0)
        emean = es_s[...] / jnp.maximum(ec_s[...][:, :64], 1.0)
        hsub = jnp.concatenate([nmean, emean], axis=1)
        z = jnp.maximum(_dot(hsub, w1_ref[...]) + b1_ref[...], 0.0)
        emb = _dot(z, w2_ref[...]) + b2_ref[...]
        emb_ref[...] = emb
        predp_ref[...] = _dot(emb[:, :64], auxw_ref[...])


def _final(hn, bn3, he, be3, w1, b1, w2, b2, auxw):
    nt = E // ECH
    full2 = lambda i: (0, 0)
    return pl.pallas_call(
        _final_body,
        grid=(nt,),
        in_specs=[
            pl.BlockSpec((N, 128), full2),
            pl.BlockSpec((1, 1, N), lambda i: (0, 0, 0)),
            pl.BlockSpec((ECH, 64), lambda i: (i, 0)),
            pl.BlockSpec((1, 1, ECH), lambda i: (i, 0, 0)),
            pl.BlockSpec((192, 256), full2),
            pl.BlockSpec((1, 256), full2),
            pl.BlockSpec((256, 128), full2),
            pl.BlockSpec((1, 128), full2),
            pl.BlockSpec((64, 128), full2),
        ],
        out_specs=[
            pl.BlockSpec((G, 128), full2),
            pl.BlockSpec((G, 128), full2),
        ],
        out_shape=[
            jax.ShapeDtypeStruct((G, 128), _f32),
            jax.ShapeDtypeStruct((G, 128), _f32),
        ],
        scratch_shapes=[
            pltpu.VMEM((G, 128), _f32),
            pltpu.VMEM((G, 128), _f32),
            pltpu.VMEM((G, 64), _f32),
            pltpu.VMEM((G, 128), _f32),
        ],
    )(hn, bn3, he, be3, w1, b1, w2, b2, auxw)


def kernel(h_node, pos_node, batch_node, h_edge, edge_index, batch_edge, params):
    src = edge_index[0]
    dst = edge_index[1]

    pos_f = pos_node.astype(_f32)
    hep = jnp.pad(h_edge.astype(_f32), ((0, 0), (0, 3)))
    weep = jnp.pad(params['edge_emb'].astype(_f32), ((0, 3), (0, 0)))

    blocks = params['blocks']
    pacs, pbs = [], []
    for blk in blocks:
        w1 = blk['We1']
        pacs.append(jnp.concatenate([w1[:128], blk['Wm'][:128]], axis=1))
        pbs.append(w1[128:256])

    hn, ac, b = _node0(h_node.astype(_f32), params['node_emb'], pacs[0], pbs[0])

    pos_pad = jnp.pad(pos_f, ((0, 0), (0, 125)))
    ps, pd = _G_POS(src, dst, pos_pad, pos_pad)
    dist, he = _prep(ps, pd, hep, weep)

    dst4 = dst.reshape(NW, EPW // GCH, 1, GCH)

    for bi, blk in enumerate(blocks):
        w1 = blk['We1']
        ac_src, b_dst = _G_ACB(src, dst, ac, b)
        he, msg = _edge(
            ac_src, b_dst, he, dist,
            w1[256:320], w1[320:321], blk['be1'].reshape(1, 128),
            blk['We2'], blk['be2'].reshape(1, 64),
            blk['Wm'][128:], blk['bm'].reshape(1, 128),
        )
        parts = _sc_scatter(dst4, msg)
        nb = min(bi + 1, 2)
        hn, ac, b = _update(hn, parts[0], parts[1], blk['Wn'][:128],
                            blk['Wn'][128:], blk['bn'].reshape(1, 128),
                            pacs[nb], pbs[nb])

    bn3 = batch_node.astype(jnp.int32).reshape(1, 1, N)
    be3 = batch_edge.astype(jnp.int32).reshape(E // ECH, 1, ECH)
    auxwp = jnp.pad(params['aux_W'].astype(_f32), ((0, 0), (0, 127)))
    emb, predp = _final(hn, bn3, he, be3,
                        params['final_W1'], params['final_b1'].reshape(1, 256),
                        params['final_W2'], params['final_b2'].reshape(1, 128),
                        auxwp)
    pred = predp[:, :1] + params['aux_b'].reshape(1, 1)
    return (emb, batch_node, pred)


# speedup vs baseline: 2.8897x; 1.0137x over previous
import functools

import jax
import jax.numpy as jnp
from jax import lax
from jax.experimental import pallas as pl
from jax.experimental.pallas import tpu as pltpu
from jax.experimental.pallas import tpu_sc as plsc

N = 10000
E = 320000
G = 64

NW = 32
EH = E // 2
EPW = EH // NW
GCH = 40
NCH = EPW // GCH
RING = 5
EPWS = E // NW

TN = 400
TEE = 1280
TEP = 2000
ECH = 8000

_f32 = jnp.float32


def _dot(a, b):
    return jnp.dot(a, b, preferred_element_type=_f32)


def _make_gather2(d1, d2):
    mesh = plsc.VectorSubcoreMesh(core_axis_name="c", subcore_axis_name="s")

    @functools.partial(
        pl.kernel,
        out_type=[jax.ShapeDtypeStruct((EH, d1), _f32),
                  jax.ShapeDtypeStruct((EH, d2), _f32)],
        mesh=mesh,
        scratch_types=[
            pltpu.VMEM((EPW,), jnp.int32),
            pltpu.VMEM((EPW,), jnp.int32),
            pltpu.VMEM((RING, GCH, d1), _f32),
            pltpu.VMEM((RING, GCH, d2), _f32),
            pltpu.SemaphoreType.DMA((RING,)),
            pltpu.SemaphoreType.DMA((RING,)),
            pltpu.SemaphoreType.DMA((RING,)),
            pltpu.SemaphoreType.DMA((RING,)),
        ])
    def kern(i1r, i2r, t1r, t2r, o1r, o2r, i1v, i2v, b1, b2, g1s, g2s,
             w1s, w2s):
        cid = lax.axis_index("c")
        sid = lax.axis_index("s")
        wid = sid * 2 + cid
        base = wid * EPW
        pltpu.sync_copy(i1r.at[pl.ds(base, EPW)], i1v)
        pltpu.sync_copy(i2r.at[pl.ds(base, EPW)], i2v)

        def fire(i, b):
            pltpu.async_copy(t1r.at[i1v.at[pl.ds(i * GCH, GCH)]], b1.at[b],
                             g1s.at[b])
            pltpu.async_copy(t2r.at[i2v.at[pl.ds(i * GCH, GCH)]], b2.at[b],
                             g2s.at[b])

        for b in range(RING):
            fire(b, b)

        def step(s, carry):
            for b in range(RING):
                i = s * RING + b
                off = base + i * GCH
                pltpu.make_async_copy(t1r.at[pl.ds(0, GCH)], b1.at[b],
                                      g1s.at[b]).wait()
                c1 = pltpu.async_copy(b1.at[b], o1r.at[pl.ds(off, GCH)],
                                      w1s.at[b])
                pltpu.make_async_copy(t2r.at[pl.ds(0, GCH)], b2.at[b],
                                      g2s.at[b]).wait()
                c2 = pltpu.async_copy(b2.at[b], o2r.at[pl.ds(off, GCH)],
                                      w2s.at[b])
                c1.wait()
                c2.wait()

                @pl.when(i + RING < NCH)
                def _():
                    fire(i + RING, b)
            return carry

        lax.fori_loop(0, NCH // RING, step, 0)

    return kern


_G_ACB = _make_gather2(256, 128)


_G_POS = _make_gather2(128, 128)


_MCH = 40
_MRING = 2
_NPT = N // 16

_sc_mesh = plsc.VectorSubcoreMesh(core_axis_name="c", subcore_axis_name="s")


@functools.partial(
    pl.kernel,
    out_type=jax.ShapeDtypeStruct((2, N, 128), _f32),
    mesh=_sc_mesh,
    scratch_types=[
        pltpu.VMEM((EPWS // GCH, 1, GCH), jnp.int32),
        pltpu.VMEM((_MRING, _MCH, 128), _f32),
        pltpu.VMEM((16, 128), _f32),
        pltpu.VMEM_SHARED((N, 128), _f32),
        pltpu.SemaphoreType.DMA((_MRING,)),
    ])
def _sc_scatter(dst4r, msgAr, msgBr, outr, idxv, mb, zb, acc, gs):
    cid = lax.axis_index("c")
    sid = lax.axis_index("s")
    wid = sid * 2 + cid

    for r in range(16):
        for c in range(8):
            zb[r, pl.ds(c * 16, 16)] = jnp.zeros((16,), _f32)
    for k in range(40):
        pltpu.sync_copy(zb, acc.at[pl.ds(sid * 624 + k * 16, 16)])
    plsc.subcore_barrier()

    pltpu.sync_copy(dst4r.at[wid], idxv)

    def stream(msgr, base):
        def fire(i, b):
            pltpu.async_copy(msgr.at[pl.ds(base + i * _MCH, _MCH)], mb.at[b],
                             gs.at[b])

        for b in range(_MRING):
            fire(b, b)

        nmch = EPWS // _MCH

        def step(s, carry):
            for b in range(_MRING):
                i = s * _MRING + b
                pltpu.make_async_copy(msgr.at[pl.ds(0, _MCH)], mb.at[b],
                                      gs.at[b]).wait()
                for j in range(_MCH // GCH):
                    pltpu.sync_copy(mb.at[b, pl.ds(j * GCH, GCH)],
                                    acc.at[idxv.at[i * (_MCH // GCH) + j, 0]],
                                    add=True)

                @pl.when(i + _MRING < nmch)
                def _():
                    fire(i + _MRING, b)
            return carry

        lax.fori_loop(0, nmch // _MRING, step, 0)

    @pl.when(wid <= 15)
    def _():
        stream(msgAr, wid * EPWS)

    @pl.when(wid >= 16)
    def _():
        stream(msgBr, (wid - 16) * EPWS)

    plsc.subcore_barrier()

    @pl.when(sid < 15)
    def _():
        pltpu.sync_copy(acc.at[pl.ds(sid * 624, 624)],
                        outr.at[cid, pl.ds(sid * 624, 624)])

    @pl.when(sid == 15)
    def _():
        pltpu.sync_copy(acc.at[pl.ds(15 * 624, 640)],
                        outr.at[cid, pl.ds(15 * 624, 640)])


def _node0_body(hnode_ref, wemb_ref, pac_ref, pb_ref, hn_ref, ac_ref, b_ref):
    hn = _dot(hnode_ref[...], wemb_ref[...])
    hn_ref[...] = hn
    ac_ref[...] = _dot(hn, pac_ref[...])
    b_ref[...] = _dot(hn, pb_ref[...])


def _node0(h_node, wemb, pac, pb):
    nt = N // TN
    full = lambda i: (0, 0)
    return pl.pallas_call(
        _node0_body,
        grid=(nt,),
        in_specs=[
            pl.BlockSpec((TN, 16), lambda i: (i, 0)),
            pl.BlockSpec((16, 128), full),
            pl.BlockSpec((128, 256), full),
            pl.BlockSpec((128, 128), full),
        ],
        out_specs=[
            pl.BlockSpec((TN, 128), lambda i: (i, 0)),
            pl.BlockSpec((TN, 256), lambda i: (i, 0)),
            pl.BlockSpec((TN, 128), lambda i: (i, 0)),
        ],
        out_shape=[
            jax.ShapeDtypeStruct((N, 128), _f32),
            jax.ShapeDtypeStruct((N, 256), _f32),
            jax.ShapeDtypeStruct((N, 128), _f32),
        ],
    )(h_node, wemb, pac, pb)


def _prep_body(ps_ref, pd_ref, hep_ref, wee_ref, dist_ref, he0_ref):
    d = ps_ref[...] - pd_ref[...]
    s = jnp.sum(d * d, axis=1, keepdims=True)
    dist_ref[...] = jnp.broadcast_to(jnp.sqrt(s), (TEP, 8))
    he0_ref[...] = _dot(hep_ref[...], wee_ref[...])


def _prep(ps, pd, hep, wee):
    nt = EH // TEP
    return pl.pallas_call(
        _prep_body,
        grid=(nt,),
        in_specs=[
            pl.BlockSpec((TEP, 128), lambda i: (i, 0)),
            pl.BlockSpec((TEP, 128), lambda i: (i, 0)),
            pl.BlockSpec((TEP, 8), lambda i: (i, 0)),
            pl.BlockSpec((8, 64), lambda i: (0, 0)),
        ],
        out_specs=[
            pl.BlockSpec((TEP, 8), lambda i: (i, 0)),
            pl.BlockSpec((TEP, 64), lambda i: (i, 0)),
        ],
        out_shape=[
            jax.ShapeDtypeStruct((EH, 8), _f32),
            jax.ShapeDtypeStruct((EH, 64), _f32),
        ],
    )(ps, pd, hep, wee)


def _edge_body(ac_ref, b_ref, he_ref, dist_ref, w1e_ref, w1v_ref, be1_ref,
               we2_ref, be2_ref, wme_ref, bm_ref, heo_ref, msg_ref):
    ac = ac_ref[...]
    dist = dist_ref[...][:, :1]
    t = ac[:, :128] + b_ref[...] + _dot(he_ref[...], w1e_ref[...])
    t = t + dist * w1v_ref[...] + be1_ref[...]
    t = jnp.maximum(t, 0.0)
    he2 = _dot(t, we2_ref[...]) + be2_ref[...]
    heo_ref[...] = he2
    msg = ac[:, 128:] + _dot(he2, wme_ref[...]) + bm_ref[...]
    msg_ref[...] = jnp.maximum(msg, 0.0)


def _edge(ac_src, b_dst, he, dist, w1e, w1v, be1, we2, be2, wme, bm):
    nt = EH // TEE
    full = lambda i: (0, 0)
    return pl.pallas_call(
        _edge_body,
        grid=(nt,),
        in_specs=[
            pl.BlockSpec((TEE, 256), lambda i: (i, 0)),
            pl.BlockSpec((TEE, 128), lambda i: (i, 0)),
            pl.BlockSpec((TEE, 64), lambda i: (i, 0)),
            pl.BlockSpec((TEE, 8), lambda i: (i, 0)),
            pl.BlockSpec((64, 128), full),
            pl.BlockSpec((1, 128), full),
            pl.BlockSpec((1, 128), full),
            pl.BlockSpec((128, 64), full),
            pl.BlockSpec((1, 64), full),
            pl.BlockSpec((64, 128), full),
            pl.BlockSpec((1, 128), full),
        ],
        out_specs=[
            pl.BlockSpec((TEE, 64), lambda i: (i, 0)),
            pl.BlockSpec((TEE, 128), lambda i: (i, 0)),
        ],
        out_shape=[
            jax.ShapeDtypeStruct((EH, 64), _f32),
            jax.ShapeDtypeStruct((EH, 128), _f32),
        ],
    )(ac_src, b_dst, he, dist, w1e, w1v, be1, we2, be2, wme, bm)


def _upd_body(hn_ref, a0_ref, a1_ref, wnh_ref, wna_ref, bn_ref, pac_ref,
              pb_ref, hn1_ref, ac_ref, b_ref):
    hn = hn_ref[...]
    agg = a0_ref[...] + a1_ref[...]
    hn1 = hn + _dot(hn, wnh_ref[...]) + _dot(agg, wna_ref[...]) + bn_ref[...]
    hn1_ref[...] = hn1
    ac_ref[...] = _dot(hn1, pac_ref[...])
    b_ref[...] = _dot(hn1, pb_ref[...])


def _update(hn, agg0, agg1, wnh, wna, bn, pac, pb):
    nt = N // TN
    full = lambda i: (0, 0)
    return pl.pallas_call(
        _upd_body,
        grid=(nt,),
        in_specs=[
            pl.BlockSpec((TN, 128), lambda i: (i, 0)),
            pl.BlockSpec((TN, 128), lambda i: (i, 0)),
            pl.BlockSpec((TN, 128), lambda i: (i, 0)),
            pl.BlockSpec((128, 128), full),
            pl.BlockSpec((128, 128), full),
            pl.BlockSpec((1, 128), full),
            pl.BlockSpec((128, 256), full),
            pl.BlockSpec((128, 128), full),
        ],
        out_specs=[
            pl.BlockSpec((TN, 128), lambda i: (i, 0)),
            pl.BlockSpec((TN, 256), lambda i: (i, 0)),
            pl.BlockSpec((TN, 128), lambda i: (i, 0)),
        ],
        out_shape=[
            jax.ShapeDtypeStruct((N, 128), _f32),
            jax.ShapeDtypeStruct((N, 256), _f32),
            jax.ShapeDtypeStruct((N, 128), _f32),
        ],
    )(hn, agg0, agg1, wnh, wna, bn, pac, pb)


def _final_body(hn_ref, bn3_ref, heA_ref, be3A_ref, heB_ref, be3B_ref,
                w1_ref, b1_ref, w2_ref, b2_ref, auxw_ref, emb_ref, predp_ref,
                ns_s, nc_s, es_s, ec_s):
    i = pl.program_id(0)
    nsteps = pl.num_programs(0)

    @pl.when(i == 0)
    def _():
        bid = bn3_ref[...].reshape(1, N)
        gid = lax.broadcasted_iota(jnp.int32, (G, N), 0)
        oh = (gid == bid).astype(_f32)
        ns_s[...] = _dot(oh, hn_ref[...])
        nc_s[...] = jnp.broadcast_to(jnp.sum(oh, axis=1, keepdims=True), (G, 128))
        es_s[...] = jnp.zeros((G, 64), _f32)
        ec_s[...] = jnp.zeros((G, 128), _f32)

    gide = lax.broadcasted_iota(jnp.int32, (G, ECH), 0)
    ohA = (gide == be3A_ref[...].reshape(1, ECH)).astype(_f32)
    ohB = (gide == be3B_ref[...].reshape(1, ECH)).astype(_f32)
    es_s[...] += _dot(ohA, heA_ref[...]) + _dot(ohB, heB_ref[...])
    ec_s[...] += jnp.broadcast_to(
        jnp.sum(ohA, axis=1, keepdims=True)
        + jnp.sum(ohB, axis=1, keepdims=True), (G, 128))

    @pl.when(i == nsteps - 1)
    def _():
        nmean = ns_s[...] / jnp.maximum(nc_s[...], 1.0)
        emean = es_s[...] / jnp.maximum(ec_s[...][:, :64], 1.0)
        hsub = jnp.concatenate([nmean, emean], axis=1)
        z = jnp.maximum(_dot(hsub, w1_ref[...]) + b1_ref[...], 0.0)
        emb = _dot(z, w2_ref[...]) + b2_ref[...]
        emb_ref[...] = emb
        predp_ref[...] = _dot(emb[:, :64], auxw_ref[...])


def _final(hn, bn3, heA, be3A, heB, be3B, w1, b1, w2, b2, auxw):
    nt = EH // ECH
    full2 = lambda i: (0, 0)
    return pl.pallas_call(
        _final_body,
        grid=(nt,),
        in_specs=[
            pl.BlockSpec((N, 128), full2),
            pl.BlockSpec((1, 1, N), lambda i: (0, 0, 0)),
            pl.BlockSpec((ECH, 64), lambda i: (i, 0)),
            pl.BlockSpec((1, 1, ECH), lambda i: (i, 0, 0)),
            pl.BlockSpec((ECH, 64), lambda i: (i, 0)),
            pl.BlockSpec((1, 1, ECH), lambda i: (i, 0, 0)),
            pl.BlockSpec((192, 256), full2),
            pl.BlockSpec((1, 256), full2),
            pl.BlockSpec((256, 128), full2),
            pl.BlockSpec((1, 128), full2),
            pl.BlockSpec((64, 128), full2),
        ],
        out_specs=[
            pl.BlockSpec((G, 128), full2),
            pl.BlockSpec((G, 128), full2),
        ],
        out_shape=[
            jax.ShapeDtypeStruct((G, 128), _f32),
            jax.ShapeDtypeStruct((G, 128), _f32),
        ],
        scratch_shapes=[
            pltpu.VMEM((G, 128), _f32),
            pltpu.VMEM((G, 128), _f32),
            pltpu.VMEM((G, 64), _f32),
            pltpu.VMEM((G, 128), _f32),
        ],
    )(hn, bn3, heA, be3A, heB, be3B, w1, b1, w2, b2, auxw)


def kernel(h_node, pos_node, batch_node, h_edge, edge_index, batch_edge, params):
    src = edge_index[0]
    dst = edge_index[1]
    srcA, srcB = src[:EH], src[EH:]
    dstA, dstB = dst[:EH], dst[EH:]

    pos_f = pos_node.astype(_f32)
    hep = jnp.pad(h_edge.astype(_f32), ((0, 0), (0, 3)))
    weep = jnp.pad(params['edge_emb'].astype(_f32), ((0, 3), (0, 0)))

    blocks = params['blocks']
    pacs, pbs = [], []
    for blk in blocks:
        w1 = blk['We1']
        pacs.append(jnp.concatenate([w1[:128], blk['Wm'][:128]], axis=1))
        pbs.append(w1[128:256])

    hn, ac, b = _node0(h_node.astype(_f32), params['node_emb'], pacs[0], pbs[0])

    pos_pad = jnp.pad(pos_f, ((0, 0), (0, 125)))
    psA, pdA = _G_POS(srcA, dstA, pos_pad, pos_pad)
    distA, heA = _prep(psA, pdA, hep[:EH], weep)
    psB, pdB = _G_POS(srcB, dstB, pos_pad, pos_pad)
    distB, heB = _prep(psB, pdB, hep[EH:], weep)

    dst4 = dst.reshape(NW, EPWS // GCH, 1, GCH)

    for bi, blk in enumerate(blocks):
        w1 = blk['We1']
        ew = (w1[256:320], w1[320:321], blk['be1'].reshape(1, 128),
              blk['We2'], blk['be2'].reshape(1, 64),
              blk['Wm'][128:], blk['bm'].reshape(1, 128))
        acA, bdA = _G_ACB(srcA, dstA, ac, b)
        heA, msgA = _edge(acA, bdA, heA, distA, *ew)
        acB, bdB = _G_ACB(srcB, dstB, ac, b)
        heB, msgB = _edge(acB, bdB, heB, distB, *ew)
        parts = _sc_scatter(dst4, msgA, msgB)
        nb = min(bi + 1, 2)
        hn, ac, b = _update(hn, parts[0], parts[1], blk['Wn'][:128],
                            blk['Wn'][128:], blk['bn'].reshape(1, 128),
                            pacs[nb], pbs[nb])

    be = batch_edge.astype(jnp.int32)
    bn3 = batch_node.astype(jnp.int32).reshape(1, 1, N)
    be3A = be[:EH].reshape(EH // ECH, 1, ECH)
    be3B = be[EH:].reshape(EH // ECH, 1, ECH)
    auxwp = jnp.pad(params['aux_W'].astype(_f32), ((0, 0), (0, 127)))
    emb, predp = _final(hn, bn3, heA, be3A, heB, be3B,
                        params['final_W1'], params['final_b1'].reshape(1, 256),
                        params['final_W2'], params['final_b2'].reshape(1, 128),
                        auxwp)
    pred = predp[:, :1] + params['aux_b'].reshape(1, 1)
    return (emb, batch_node, pred)


# speedup vs baseline: 2.9167x; 1.0093x over previous
import functools

import jax
import jax.numpy as jnp
from jax import lax
from jax.experimental import pallas as pl
from jax.experimental.pallas import tpu as pltpu
from jax.experimental.pallas import tpu_sc as plsc

N = 10000
E = 320000
G = 64

NW = 32
EH = E // 2
EPW = EH // NW
GCH = 40
NCH = EPW // GCH
RING = 5
EPWS = E // NW

TN = 400
TEE = 1280
TEP = 2000
ECH = 8000

_f32 = jnp.float32


def _dot(a, b):
    return jnp.dot(a, b, preferred_element_type=_f32)


def _make_gather2(d1, d2):
    mesh = plsc.VectorSubcoreMesh(core_axis_name="c", subcore_axis_name="s")

    @functools.partial(
        pl.kernel,
        out_type=[jax.ShapeDtypeStruct((EH, d1), _f32),
                  jax.ShapeDtypeStruct((EH, d2), _f32)],
        mesh=mesh,
        scratch_types=[
            pltpu.VMEM((EPW,), jnp.int32),
            pltpu.VMEM((EPW,), jnp.int32),
            pltpu.VMEM((RING, GCH, d1), _f32),
            pltpu.VMEM((RING, GCH, d2), _f32),
            pltpu.SemaphoreType.DMA((RING,)),
            pltpu.SemaphoreType.DMA((RING,)),
            pltpu.SemaphoreType.DMA((RING,)),
            pltpu.SemaphoreType.DMA((RING,)),
        ])
    def kern(i1r, i2r, t1r, t2r, o1r, o2r, i1v, i2v, b1, b2, g1s, g2s,
             w1s, w2s):
        cid = lax.axis_index("c")
        sid = lax.axis_index("s")
        wid = sid * 2 + cid
        base = wid * EPW
        pltpu.sync_copy(i1r.at[pl.ds(base, EPW)], i1v)
        pltpu.sync_copy(i2r.at[pl.ds(base, EPW)], i2v)

        def fire(i, b):
            pltpu.async_copy(t1r.at[i1v.at[pl.ds(i * GCH, GCH)]], b1.at[b],
                             g1s.at[b])
            pltpu.async_copy(t2r.at[i2v.at[pl.ds(i * GCH, GCH)]], b2.at[b],
                             g2s.at[b])

        for b in range(RING):
            fire(b, b)

        def step(s, carry):
            for b in range(RING):
                i = s * RING + b
                off = base + i * GCH
                pltpu.make_async_copy(t1r.at[pl.ds(0, GCH)], b1.at[b],
                                      g1s.at[b]).wait()
                c1 = pltpu.async_copy(b1.at[b], o1r.at[pl.ds(off, GCH)],
                                      w1s.at[b])
                pltpu.make_async_copy(t2r.at[pl.ds(0, GCH)], b2.at[b],
                                      g2s.at[b]).wait()
                c2 = pltpu.async_copy(b2.at[b], o2r.at[pl.ds(off, GCH)],
                                      w2s.at[b])
                c1.wait()
                c2.wait()

                @pl.when(i + RING < NCH)
                def _():
                    fire(i + RING, b)
            return carry

        lax.fori_loop(0, NCH // RING, step, 0)

    return kern


_G_ACB = _make_gather2(256, 128)


_G_POS = _make_gather2(128, 128)


_MCH = 40
_MRING = 2
_NPT = N // 16

_sc_mesh = plsc.VectorSubcoreMesh(core_axis_name="c", subcore_axis_name="s")


@functools.partial(
    pl.kernel,
    out_type=jax.ShapeDtypeStruct((2, N, 128), _f32),
    mesh=_sc_mesh,
    scratch_types=[
        pltpu.VMEM((EPWS // GCH, 1, GCH), jnp.int32),
        pltpu.VMEM((_MRING, _MCH, 128), _f32),
        pltpu.VMEM((16, 128), _f32),
        pltpu.VMEM_SHARED((N, 128), _f32),
        pltpu.SemaphoreType.DMA((_MRING,)),
    ])
def _sc_scatter(dst4r, msgAr, msgBr, outr, idxv, mb, zb, acc, gs):
    cid = lax.axis_index("c")
    sid = lax.axis_index("s")
    wid = sid * 2 + cid

    for r in range(16):
        for c in range(8):
            zb[r, pl.ds(c * 16, 16)] = jnp.zeros((16,), _f32)
    for k in range(40):
        pltpu.sync_copy(zb, acc.at[pl.ds(sid * 624 + k * 16, 16)])
    plsc.subcore_barrier()

    pltpu.sync_copy(dst4r.at[wid], idxv)

    def stream(msgr, base):
        def fire(i, b):
            pltpu.async_copy(msgr.at[pl.ds(base + i * _MCH, _MCH)], mb.at[b],
                             gs.at[b])

        for b in range(_MRING):
            fire(b, b)

        nmch = EPWS // _MCH

        def step(s, carry):
            for b in range(_MRING):
                i = s * _MRING + b
                pltpu.make_async_copy(msgr.at[pl.ds(0, _MCH)], mb.at[b],
                                      gs.at[b]).wait()
                for j in range(_MCH // GCH):
                    pltpu.sync_copy(mb.at[b, pl.ds(j * GCH, GCH)],
                                    acc.at[idxv.at[i * (_MCH // GCH) + j, 0]],
                                    add=True)

                @pl.when(i + _MRING < nmch)
                def _():
                    fire(i + _MRING, b)
            return carry

        lax.fori_loop(0, nmch // _MRING, step, 0)

    @pl.when(wid <= 15)
    def _():
        stream(msgAr, wid * EPWS)

    @pl.when(wid >= 16)
    def _():
        stream(msgBr, (wid - 16) * EPWS)

    plsc.subcore_barrier()

    @pl.when(sid < 15)
    def _():
        pltpu.sync_copy(acc.at[pl.ds(sid * 624, 624)],
                        outr.at[cid, pl.ds(sid * 624, 624)])

    @pl.when(sid == 15)
    def _():
        pltpu.sync_copy(acc.at[pl.ds(15 * 624, 640)],
                        outr.at[cid, pl.ds(15 * 624, 640)])


def _node0_body(hnode_ref, wemb_ref, pac_ref, pb_ref, hn_ref, ac_ref, b_ref):
    hn = _dot(hnode_ref[...], wemb_ref[...])
    hn_ref[...] = hn
    ac_ref[...] = _dot(hn, pac_ref[...])
    b_ref[...] = _dot(hn, pb_ref[...])


def _node0(h_node, wemb, pac, pb):
    nt = N // TN
    full = lambda i: (0, 0)
    return pl.pallas_call(
        _node0_body,
        grid=(nt,),
        in_specs=[
            pl.BlockSpec((TN, 16), lambda i: (i, 0)),
            pl.BlockSpec((16, 128), full),
            pl.BlockSpec((128, 256), full),
            pl.BlockSpec((128, 128), full),
        ],
        out_specs=[
            pl.BlockSpec((TN, 128), lambda i: (i, 0)),
            pl.BlockSpec((TN, 256), lambda i: (i, 0)),
            pl.BlockSpec((TN, 128), lambda i: (i, 0)),
        ],
        out_shape=[
            jax.ShapeDtypeStruct((N, 128), _f32),
            jax.ShapeDtypeStruct((N, 256), _f32),
            jax.ShapeDtypeStruct((N, 128), _f32),
        ],
    )(h_node, wemb, pac, pb)


def _prep_body(ps_ref, pd_ref, hep_ref, wee_ref, dist_ref, he0_ref):
    d = ps_ref[...] - pd_ref[...]
    s = jnp.sum(d * d, axis=1, keepdims=True)
    dist_ref[...] = jnp.broadcast_to(jnp.sqrt(s), (TEP, 8))
    he0_ref[...] = _dot(hep_ref[...], wee_ref[...])


def _prep(ps, pd, hep, wee):
    nt = EH // TEP
    return pl.pallas_call(
        _prep_body,
        grid=(nt,),
        in_specs=[
            pl.BlockSpec((TEP, 128), lambda i: (i, 0)),
            pl.BlockSpec((TEP, 128), lambda i: (i, 0)),
            pl.BlockSpec((TEP, 8), lambda i: (i, 0)),
            pl.BlockSpec((8, 128), lambda i: (0, 0)),
        ],
        out_specs=[
            pl.BlockSpec((TEP, 8), lambda i: (i, 0)),
            pl.BlockSpec((TEP, 128), lambda i: (i, 0)),
        ],
        out_shape=[
            jax.ShapeDtypeStruct((EH, 8), _f32),
            jax.ShapeDtypeStruct((EH, 128), _f32),
        ],
    )(ps, pd, hep, wee)


def _edge_body_first(ac_ref, b_ref, tp_ref, dist_ref, w1v_ref, be1_ref,
                     we2m_ref, bm2_ref, t_ref, msg_ref):
    ac = ac_ref[...]
    dist = dist_ref[...][:, :1]
    t = ac[:, :128] + b_ref[...] + tp_ref[...]
    t = t + dist * w1v_ref[...] + be1_ref[...]
    t = jnp.maximum(t, 0.0)
    t_ref[...] = t
    msg = ac[:, 128:] + _dot(t, we2m_ref[...]) + bm2_ref[...]
    msg_ref[...] = jnp.maximum(msg, 0.0)


def _edge_body_mm(ac_ref, b_ref, tp_ref, dist_ref, w2w1_ref, w1v_ref,
                  be1_ref, we2m_ref, bm2_ref, t_ref, msg_ref):
    ac = ac_ref[...]
    dist = dist_ref[...][:, :1]
    t = ac[:, :128] + b_ref[...] + _dot(tp_ref[...], w2w1_ref[...])
    t = t + dist * w1v_ref[...] + be1_ref[...]
    t = jnp.maximum(t, 0.0)
    t_ref[...] = t
    msg = ac[:, 128:] + _dot(t, we2m_ref[...]) + bm2_ref[...]
    msg_ref[...] = jnp.maximum(msg, 0.0)


def _edge(ac_src, b_dst, tp, dist, w2w1, w1v, be1, we2m, bm2):
    nt = EH // TEE
    full = lambda i: (0, 0)
    w_specs = [
        pl.BlockSpec((1, 128), full),
        pl.BlockSpec((1, 128), full),
        pl.BlockSpec((128, 128), full),
        pl.BlockSpec((1, 128), full),
    ]
    args = [ac_src, b_dst, tp, dist]
    if w2w1 is None:
        body = _edge_body_first
    else:
        body = _edge_body_mm
        w_specs = [pl.BlockSpec((128, 128), full)] + w_specs
        args.append(w2w1)
    args += [w1v, be1, we2m, bm2]
    return pl.pallas_call(
        body,
        grid=(nt,),
        in_specs=[
            pl.BlockSpec((TEE, 256), lambda i: (i, 0)),
            pl.BlockSpec((TEE, 128), lambda i: (i, 0)),
            pl.BlockSpec((TEE, 128), lambda i: (i, 0)),
            pl.BlockSpec((TEE, 8), lambda i: (i, 0)),
        ] + w_specs,
        out_specs=[
            pl.BlockSpec((TEE, 128), lambda i: (i, 0)),
            pl.BlockSpec((TEE, 128), lambda i: (i, 0)),
        ],
        out_shape=[
            jax.ShapeDtypeStruct((EH, 128), _f32),
            jax.ShapeDtypeStruct((EH, 128), _f32),
        ],
    )(*args)


def _upd_body(hn_ref, a0_ref, a1_ref, wnh_ref, wna_ref, bn_ref, pac_ref,
              pb_ref, hn1_ref, ac_ref, b_ref):
    hn = hn_ref[...]
    agg = a0_ref[...] + a1_ref[...]
    hn1 = hn + _dot(hn, wnh_ref[...]) + _dot(agg, wna_ref[...]) + bn_ref[...]
    hn1_ref[...] = hn1
    ac_ref[...] = _dot(hn1, pac_ref[...])
    b_ref[...] = _dot(hn1, pb_ref[...])


def _update(hn, agg0, agg1, wnh, wna, bn, pac, pb):
    nt = N // TN
    full = lambda i: (0, 0)
    return pl.pallas_call(
        _upd_body,
        grid=(nt,),
        in_specs=[
            pl.BlockSpec((TN, 128), lambda i: (i, 0)),
            pl.BlockSpec((TN, 128), lambda i: (i, 0)),
            pl.BlockSpec((TN, 128), lambda i: (i, 0)),
            pl.BlockSpec((128, 128), full),
            pl.BlockSpec((128, 128), full),
            pl.BlockSpec((1, 128), full),
            pl.BlockSpec((128, 256), full),
            pl.BlockSpec((128, 128), full),
        ],
        out_specs=[
            pl.BlockSpec((TN, 128), lambda i: (i, 0)),
            pl.BlockSpec((TN, 256), lambda i: (i, 0)),
            pl.BlockSpec((TN, 128), lambda i: (i, 0)),
        ],
        out_shape=[
            jax.ShapeDtypeStruct((N, 128), _f32),
            jax.ShapeDtypeStruct((N, 256), _f32),
            jax.ShapeDtypeStruct((N, 128), _f32),
        ],
    )(hn, agg0, agg1, wnh, wna, bn, pac, pb)


def _final_body(hn_ref, bn3_ref, heA_ref, be3A_ref, heB_ref, be3B_ref,
                we2_ref, be2_ref, w1_ref, b1_ref, w2_ref, b2_ref, auxw_ref,
                emb_ref, predp_ref, ns_s, nc_s, es_s, ec_s):
    i = pl.program_id(0)
    nsteps = pl.num_programs(0)

    @pl.when(i == 0)
    def _():
        bid = bn3_ref[...].reshape(1, N)
        gid = lax.broadcasted_iota(jnp.int32, (G, N), 0)
        oh = (gid == bid).astype(_f32)
        ns_s[...] = _dot(oh, hn_ref[...])
        nc_s[...] = jnp.broadcast_to(jnp.sum(oh, axis=1, keepdims=True), (G, 128))
        es_s[...] = jnp.zeros((G, 128), _f32)
        ec_s[...] = jnp.zeros((G, 128), _f32)

    gide = lax.broadcasted_iota(jnp.int32, (G, ECH), 0)
    ohA = (gide == be3A_ref[...].reshape(1, ECH)).astype(_f32)
    ohB = (gide == be3B_ref[...].reshape(1, ECH)).astype(_f32)
    es_s[...] += _dot(ohA, heA_ref[...]) + _dot(ohB, heB_ref[...])
    ec_s[...] += jnp.broadcast_to(
        jnp.sum(ohA, axis=1, keepdims=True)
        + jnp.sum(ohB, axis=1, keepdims=True), (G, 128))

    @pl.when(i == nsteps - 1)
    def _():
        nmean = ns_s[...] / jnp.maximum(nc_s[...], 1.0)
        emean = (_dot(es_s[...], we2_ref[...])
                 / jnp.maximum(ec_s[...][:, :64], 1.0) + be2_ref[...])
        hsub = jnp.concatenate([nmean, emean], axis=1)
        z = jnp.maximum(_dot(hsub, w1_ref[...]) + b1_ref[...], 0.0)
        emb = _dot(z, w2_ref[...]) + b2_ref[...]
        emb_ref[...] = emb
        predp_ref[...] = _dot(emb[:, :64], auxw_ref[...])


def _final(hn, bn3, heA, be3A, heB, be3B, we2, be2, w1, b1, w2, b2, auxw):
    nt = EH // ECH
    full2 = lambda i: (0, 0)
    return pl.pallas_call(
        _final_body,
        grid=(nt,),
        in_specs=[
            pl.BlockSpec((N, 128), full2),
            pl.BlockSpec((1, 1, N), lambda i: (0, 0, 0)),
            pl.BlockSpec((ECH, 128), lambda i: (i, 0)),
            pl.BlockSpec((1, 1, ECH), lambda i: (i, 0, 0)),
            pl.BlockSpec((ECH, 128), lambda i: (i, 0)),
            pl.BlockSpec((1, 1, ECH), lambda i: (i, 0, 0)),
            pl.BlockSpec((128, 64), full2),
            pl.BlockSpec((1, 64), full2),
            pl.BlockSpec((192, 256), full2),
            pl.BlockSpec((1, 256), full2),
            pl.BlockSpec((256, 128), full2),
            pl.BlockSpec((1, 128), full2),
            pl.BlockSpec((64, 128), full2),
        ],
        out_specs=[
            pl.BlockSpec((G, 128), full2),
            pl.BlockSpec((G, 128), full2),
        ],
        out_shape=[
            jax.ShapeDtypeStruct((G, 128), _f32),
            jax.ShapeDtypeStruct((G, 128), _f32),
        ],
        scratch_shapes=[
            pltpu.VMEM((G, 128), _f32),
            pltpu.VMEM((G, 128), _f32),
            pltpu.VMEM((G, 128), _f32),
            pltpu.VMEM((G, 128), _f32),
        ],
    )(hn, bn3, heA, be3A, heB, be3B, we2, be2, w1, b1, w2, b2, auxw)


def kernel(h_node, pos_node, batch_node, h_edge, edge_index, batch_edge, params):
    src = edge_index[0]
    dst = edge_index[1]
    srcA, srcB = src[:EH], src[EH:]
    dstA, dstB = dst[:EH], dst[EH:]

    pos_f = pos_node.astype(_f32)
    hep = jnp.pad(h_edge.astype(_f32), ((0, 0), (0, 3)))
    weep = jnp.pad(params['edge_emb'].astype(_f32), ((0, 3), (0, 0)))

    blocks = params['blocks']
    pacs, pbs = [], []
    for blk in blocks:
        w1 = blk['We1']
        pacs.append(jnp.concatenate([w1[:128], blk['Wm'][:128]], axis=1))
        pbs.append(w1[128:256])

    w1es = [blk['We1'][256:320] for blk in blocks]
    wme = [blk['Wm'][128:] for blk in blocks]
    wee1 = _dot(weep, w1es[0])
    we2ms = [_dot(blk['We2'], wme[i]) for i, blk in enumerate(blocks)]
    bm2s = [_dot(blk['be2'].reshape(1, 64), wme[i]) + blk['bm'].reshape(1, 128)
            for i, blk in enumerate(blocks)]
    w2w1s = [None,
             _dot(blocks[0]['We2'], w1es[1]),
             _dot(blocks[1]['We2'], w1es[2])]
    be1es = [blocks[0]['be1'].reshape(1, 128),
             blocks[1]['be1'].reshape(1, 128)
             + _dot(blocks[0]['be2'].reshape(1, 64), w1es[1]),
             blocks[2]['be1'].reshape(1, 128)
             + _dot(blocks[1]['be2'].reshape(1, 64), w1es[2])]

    hn, ac, b = _node0(h_node.astype(_f32), params['node_emb'], pacs[0], pbs[0])

    pos_pad = jnp.pad(pos_f, ((0, 0), (0, 125)))
    psA, pdA = _G_POS(srcA, dstA, pos_pad, pos_pad)
    distA, tA = _prep(psA, pdA, hep[:EH], wee1)
    psB, pdB = _G_POS(srcB, dstB, pos_pad, pos_pad)
    distB, tB = _prep(psB, pdB, hep[EH:], wee1)

    dst4 = dst.reshape(NW, EPWS // GCH, 1, GCH)

    for bi, blk in enumerate(blocks):
        ew = (w2w1s[bi], blk['We1'][320:321], be1es[bi], we2ms[bi], bm2s[bi])
        acA, bdA = _G_ACB(srcA, dstA, ac, b)
        tA, msgA = _edge(acA, bdA, tA, distA, *ew)
        acB, bdB = _G_ACB(srcB, dstB, ac, b)
        tB, msgB = _edge(acB, bdB, tB, distB, *ew)
        parts = _sc_scatter(dst4, msgA, msgB)
        nb = min(bi + 1, 2)
        hn, ac, b = _update(hn, parts[0], parts[1], blk['Wn'][:128],
                            blk['Wn'][128:], blk['bn'].reshape(1, 128),
                            pacs[nb], pbs[nb])

    be = batch_edge.astype(jnp.int32)
    bn3 = batch_node.astype(jnp.int32).reshape(1, 1, N)
    be3A = be[:EH].reshape(EH // ECH, 1, ECH)
    be3B = be[EH:].reshape(EH // ECH, 1, ECH)
    auxwp = jnp.pad(params['aux_W'].astype(_f32), ((0, 0), (0, 127)))
    emb, predp = _final(hn, bn3, tA, be3A, tB, be3B,
                        blocks[2]['We2'], blocks[2]['be2'].reshape(1, 64),
                        params['final_W1'], params['final_b1'].reshape(1, 256),
                        params['final_W2'], params['final_b2'].reshape(1, 128),
                        auxwp)
    pred = predp[:, :1] + params['aux_b'].reshape(1, 1)
    return (emb, batch_node, pred)
